# Initial kernel scaffold; baseline (speedup 1.0000x reference)
import jax
import jax.numpy as jnp
from jax.experimental import pallas as pl


def kernel(x, batch, W_enc, b_enc, W1, b1, gamma1, beta1, W2, b2):
    raise NotImplementedError("write your pallas kernel here")


# speedup vs baseline: 1.7834x; 1.7834x over previous
import functools

import jax
import jax.numpy as jnp
from jax import lax
from jax.experimental import pallas as pl
from jax.experimental.pallas import tpu as pltpu

N = 320000
D = 128
H = 256
NT = 10
S = 512

BLK = 2560
WIN = 64
NEG = float("-inf")


def _shift_down(a, k, fill):
    pad = jnp.full((k,) + a.shape[1:], fill, a.dtype)
    return jnp.concatenate([pad, a[: a.shape[0] - k]], axis=0)


def _fused_kernel(x_ref, bcol_ref, blane_ref, wenc_ref, benc_ref,
                  w1_ref, b1_ref, g1_ref, be1_ref, w2_ref, b2_ref,
                  out_ref, acc_ref, *, nblk):
    i = pl.program_id(0)

    @pl.when(i == 0)
    def _init():
        acc_ref[...] = jnp.full((S, D), NEG, jnp.float32)

    emb = jnp.dot(x_ref[...], wenc_ref[...],
                  preferred_element_type=jnp.float32)

    bcol = bcol_ref[...]
    k = 1
    while k < BLK:
        prev_b = _shift_down(bcol, k, -1)
        prev_e = _shift_down(emb, k, NEG)
        same = prev_b == bcol
        emb = jnp.where(same, jnp.maximum(emb, prev_e), emb)
        k *= 2

    blane = blane_ref[0]
    nxt = jnp.concatenate(
        [blane[:, 1:], jnp.full((1, 1), -1, jnp.int32)], axis=1)
    is_end = blane != nxt

    b_first = jnp.min(blane)
    b_last = jnp.max(blane)

    def w_body(w0):
        seg = lax.broadcasted_iota(jnp.int32, (WIN, BLK), 0) + w0
        sel = (jnp.broadcast_to(blane, (WIN, BLK)) == seg) & \
            jnp.broadcast_to(is_end, (WIN, BLK))
        m = jnp.where(sel, 1.0, 0.0).astype(jnp.float32)
        contrib = jnp.dot(m, emb, preferred_element_type=jnp.float32)
        present = jnp.max(m, axis=1, keepdims=True) > 0.0
        contrib = jnp.where(present, contrib, NEG)
        cur = acc_ref[pl.ds(w0, WIN), :]
        acc_ref[pl.ds(w0, WIN), :] = jnp.maximum(cur, contrib)
        rest = jnp.min(jnp.where(blane >= w0 + WIN, blane, jnp.int32(2**30)))
        return (rest // WIN) * WIN

    lax.while_loop(lambda w0: w0 <= b_last, w_body, (b_first // WIN) * WIN)

    @pl.when(i == nblk - 1)
    def _mlp():
        g = acc_ref[...] + benc_ref[...]
        h = jnp.dot(g, w1_ref[...],
                    preferred_element_type=jnp.float32) + b1_ref[...]
        mu = jnp.mean(h, axis=-1, keepdims=True)
        var = jnp.mean((h - mu) * (h - mu), axis=-1, keepdims=True)
        h = (h - mu) * lax.rsqrt(var + 1e-5) * g1_ref[...] + be1_ref[...]
        h = jnp.maximum(h, 0.0)
        out_ref[...] = jnp.dot(h, w2_ref[...],
                               preferred_element_type=jnp.float32) + b2_ref[...]


@functools.partial(jax.jit, static_argnames=("interpret",))
def kernel(x, batch, W_enc, b_enc, W1, b1, gamma1, beta1, W2, b2,
           interpret=False):
    n, d = x.shape
    nblk = n // BLK
    assert nblk * BLK == n
    bcol = batch.reshape(n, 1)
    blane = batch.reshape(nblk, 1, BLK)

    grid_spec = pltpu.PrefetchScalarGridSpec(
        num_scalar_prefetch=0,
        grid=(nblk,),
        in_specs=[
            pl.BlockSpec((BLK, d), lambda i: (i, 0)),
            pl.BlockSpec((BLK, 1), lambda i: (i, 0)),
            pl.BlockSpec((1, 1, BLK), lambda i: (i, 0, 0)),
            pl.BlockSpec((d, D), lambda i: (0, 0)),
            pl.BlockSpec((1, D), lambda i: (0, 0)),
            pl.BlockSpec((D, H), lambda i: (0, 0)),
            pl.BlockSpec((1, H), lambda i: (0, 0)),
            pl.BlockSpec((1, H), lambda i: (0, 0)),
            pl.BlockSpec((1, H), lambda i: (0, 0)),
            pl.BlockSpec((H, NT), lambda i: (0, 0)),
            pl.BlockSpec((1, NT), lambda i: (0, 0)),
        ],
        out_specs=pl.BlockSpec((S, NT), lambda i: (0, 0)),
        scratch_shapes=[pltpu.VMEM((S, D), jnp.float32)],
    )
    return pl.pallas_call(
        functools.partial(_fused_kernel, nblk=nblk),
        grid_spec=grid_spec,
        out_shape=jax.ShapeDtypeStruct((S, NT), jnp.float32),
        compiler_params=pltpu.CompilerParams(
            dimension_semantics=("arbitrary",)),
        interpret=interpret,
    )(x, bcol, blane, W_enc, b_enc.reshape(1, D), W1, b1.reshape(1, H),
      gamma1.reshape(1, H), beta1.reshape(1, H), W2, b2.reshape(1, NT))


# speedup vs baseline: 2.2571x; 1.2656x over previous
import functools

import jax
import jax.numpy as jnp
from jax import lax
from jax.experimental import pallas as pl
from jax.experimental.pallas import tpu as pltpu

N = 320000
D = 128
H = 256
NT = 10
S = 512

BLK = 2560
WIN = 16
NEG = float("-inf")


def _shift_down(a, k, fill):
    pad = jnp.full((k,) + a.shape[1:], fill, a.dtype)
    return jnp.concatenate([pad, a[: a.shape[0] - k]], axis=0)


def _shift_mid(a, k):
    pad = jnp.full((a.shape[0], k, a.shape[2]), NEG, a.dtype)
    return jnp.concatenate([pad, a[:, : a.shape[1] - k, :]], axis=1)


def _fused_kernel(x_ref, bcol_ref, blane_ref, wenc_ref, benc_ref,
                  w1_ref, b1_ref, g1_ref, be1_ref, w2_ref, b2_ref,
                  out_ref, acc_ref, *, nblk):
    i = pl.program_id(0)

    @pl.when(i == 0)
    def _init():
        acc_ref[...] = jnp.full((S, D), NEG, jnp.float32)

    emb = jnp.dot(x_ref[...], wenc_ref[...],
                  preferred_element_type=jnp.float32)

    blane = blane_ref[0]
    b_first = jnp.min(blane)
    b_last = jnp.max(blane)

    rng = jnp.max(emb) - jnp.min(emb)
    off = rng * 1.0009765625 + 1.0
    lbf = (bcol_ref[...] - b_first).astype(jnp.float32)
    z = emb + lbf * off

    nch = BLK // 8
    z3 = z.reshape(nch, 8, D)
    for k in (1, 2, 4):
        z3 = jnp.maximum(z3, _shift_mid(z3, k))
    ctot = z3[:, 7, :]
    k = 1
    while k < nch:
        ctot = jnp.maximum(ctot, _shift_down(ctot, k, NEG))
        k *= 2
    carry = _shift_down(ctot, 1, NEG).reshape(nch, 1, D)
    zf = jnp.maximum(z3, carry).reshape(BLK, D)

    nxt = jnp.concatenate(
        [blane[:, 1:], jnp.full((1, 1), -1, jnp.int32)], axis=1)
    key = jnp.where(blane != nxt, blane, -1)

    def w_body(w0):
        seg = lax.broadcasted_iota(jnp.int32, (WIN, BLK), 0) + w0
        m = jnp.where(jnp.broadcast_to(key, (WIN, BLK)) == seg, 1.0, 0.0)
        contrib = jnp.dot(m.astype(jnp.float32), zf,
                          preferred_element_type=jnp.float32)
        woff = (lax.broadcasted_iota(jnp.int32, (WIN, 1), 0)
                + (w0 - b_first)).astype(jnp.float32) * off
        present = jnp.max(m, axis=1, keepdims=True) > 0.0
        contrib = jnp.where(present, contrib - woff, NEG)
        cur = acc_ref[pl.ds(w0, WIN), :]
        acc_ref[pl.ds(w0, WIN), :] = jnp.maximum(cur, contrib)
        rest = jnp.min(jnp.where(blane >= w0 + WIN, blane, jnp.int32(2**30)))
        return (rest // WIN) * WIN

    lax.while_loop(lambda w0: w0 <= b_last, w_body, (b_first // WIN) * WIN)

    @pl.when(i == nblk - 1)
    def _mlp():
        g = acc_ref[...] + benc_ref[...]
        h = jnp.dot(g, w1_ref[...],
                    preferred_element_type=jnp.float32) + b1_ref[...]
        mu = jnp.mean(h, axis=-1, keepdims=True)
        var = jnp.mean((h - mu) * (h - mu), axis=-1, keepdims=True)
        h = (h - mu) * lax.rsqrt(var + 1e-5) * g1_ref[...] + be1_ref[...]
        h = jnp.maximum(h, 0.0)
        out_ref[...] = jnp.dot(h, w2_ref[...],
                               preferred_element_type=jnp.float32) + b2_ref[...]


@functools.partial(jax.jit, static_argnames=("interpret",))
def kernel(x, batch, W_enc, b_enc, W1, b1, gamma1, beta1, W2, b2,
           interpret=False):
    n, d = x.shape
    nblk = n // BLK
    assert nblk * BLK == n
    bcol = batch.reshape(n, 1)
    blane = batch.reshape(nblk, 1, BLK)

    grid_spec = pltpu.PrefetchScalarGridSpec(
        num_scalar_prefetch=0,
        grid=(nblk,),
        in_specs=[
            pl.BlockSpec((BLK, d), lambda i: (i, 0)),
            pl.BlockSpec((BLK, 1), lambda i: (i, 0)),
            pl.BlockSpec((1, 1, BLK), lambda i: (i, 0, 0)),
            pl.BlockSpec((d, D), lambda i: (0, 0)),
            pl.BlockSpec((1, D), lambda i: (0, 0)),
            pl.BlockSpec((D, H), lambda i: (0, 0)),
            pl.BlockSpec((1, H), lambda i: (0, 0)),
            pl.BlockSpec((1, H), lambda i: (0, 0)),
            pl.BlockSpec((1, H), lambda i: (0, 0)),
            pl.BlockSpec((H, NT), lambda i: (0, 0)),
            pl.BlockSpec((1, NT), lambda i: (0, 0)),
        ],
        out_specs=pl.BlockSpec((S, NT), lambda i: (0, 0)),
        scratch_shapes=[pltpu.VMEM((S, D), jnp.float32)],
    )
    return pl.pallas_call(
        functools.partial(_fused_kernel, nblk=nblk),
        grid_spec=grid_spec,
        out_shape=jax.ShapeDtypeStruct((S, NT), jnp.float32),
        compiler_params=pltpu.CompilerParams(
            dimension_semantics=("arbitrary",)),
        interpret=interpret,
    )(x, bcol, blane, W_enc, b_enc.reshape(1, D), W1, b1.reshape(1, H),
      gamma1.reshape(1, H), beta1.reshape(1, H), W2, b2.reshape(1, NT))


# speedup vs baseline: 3.3385x; 1.4791x over previous
import functools

import jax
import jax.numpy as jnp
from jax import lax
from jax.experimental import pallas as pl
from jax.experimental.pallas import tpu as pltpu

N = 320000
D = 128
H = 256
NT = 10
S = 512

BLK = 2560
WIN = 64
NEG = float("-inf")


def _shift_down(a, k, fill):
    pad = jnp.full((k,) + a.shape[1:], fill, a.dtype)
    return jnp.concatenate([pad, a[: a.shape[0] - k]], axis=0)


def _shift_mid(a, k):
    pad = jnp.full((a.shape[0], k, a.shape[2]), NEG, a.dtype)
    return jnp.concatenate([pad, a[:, : a.shape[1] - k, :]], axis=1)


def _fused_kernel(x_ref, bcol_ref, blane_ref, wenc_ref, benc_ref,
                  w1_ref, b1_ref, g1_ref, be1_ref, w2_ref, b2_ref,
                  out_ref, acc_ref, *, nblk):
    i = pl.program_id(0)

    @pl.when(i == 0)
    def _init():
        acc_ref[...] = jnp.full((S, D), NEG, jnp.float32)

    emb = jnp.dot(x_ref[...], wenc_ref[...],
                  preferred_element_type=jnp.float32)

    blane = blane_ref[0]
    b_first = jnp.min(blane)
    b_last = jnp.max(blane)

    rng = (jnp.max(emb, axis=0, keepdims=True)
           - jnp.min(emb, axis=0, keepdims=True))
    off = rng * 1.5 + 1.0
    lbf = (bcol_ref[...] - b_first).astype(jnp.float32)
    enc = lbf * off
    z = emb + enc

    zf = z
    k = 1
    while k < BLK:
        zf = jnp.maximum(zf, _shift_down(zf, k, NEG))
        k *= 2
    zf = zf - enc

    nxt = jnp.concatenate(
        [blane[:, 1:], jnp.full((1, 1), -1, jnp.int32)], axis=1)
    key = jnp.where(blane != nxt, blane, -1)

    def w_body(w0):
        seg = lax.broadcasted_iota(jnp.int32, (WIN, BLK), 0) + w0
        m = jnp.where(jnp.broadcast_to(key, (WIN, BLK)) == seg, 1.0, 0.0)
        contrib = jnp.dot(m.astype(jnp.float32), zf,
                          preferred_element_type=jnp.float32)
        present = jnp.max(m, axis=1, keepdims=True) > 0.0
        contrib = jnp.where(present, contrib, NEG)
        cur = acc_ref[pl.ds(w0, WIN), :]
        acc_ref[pl.ds(w0, WIN), :] = jnp.maximum(cur, contrib)
        rest = jnp.min(jnp.where(blane >= w0 + WIN, blane, jnp.int32(2**30)))
        return (rest // WIN) * WIN

    lax.while_loop(lambda w0: w0 <= b_last, w_body, (b_first // WIN) * WIN)

    @pl.when(i == nblk - 1)
    def _mlp():
        g = acc_ref[...] + benc_ref[...]
        h = jnp.dot(g, w1_ref[...],
                    preferred_element_type=jnp.float32) + b1_ref[...]
        mu = jnp.mean(h, axis=-1, keepdims=True)
        var = jnp.mean((h - mu) * (h - mu), axis=-1, keepdims=True)
        h = (h - mu) * lax.rsqrt(var + 1e-5) * g1_ref[...] + be1_ref[...]
        h = jnp.maximum(h, 0.0)
        out_ref[...] = jnp.dot(h, w2_ref[...],
                               preferred_element_type=jnp.float32) + b2_ref[...]


@functools.partial(jax.jit, static_argnames=("interpret",))
def kernel(x, batch, W_enc, b_enc, W1, b1, gamma1, beta1, W2, b2,
           interpret=False):
    n, d = x.shape
    nblk = n // BLK
    assert nblk * BLK == n
    bcol = batch.reshape(n, 1)
    blane = batch.reshape(nblk, 1, BLK)

    grid_spec = pltpu.PrefetchScalarGridSpec(
        num_scalar_prefetch=0,
        grid=(nblk,),
        in_specs=[
            pl.BlockSpec((BLK, d), lambda i: (i, 0)),
            pl.BlockSpec((BLK, 1), lambda i: (i, 0)),
            pl.BlockSpec((1, 1, BLK), lambda i: (i, 0, 0)),
            pl.BlockSpec((d, D), lambda i: (0, 0)),
            pl.BlockSpec((1, D), lambda i: (0, 0)),
            pl.BlockSpec((D, H), lambda i: (0, 0)),
            pl.BlockSpec((1, H), lambda i: (0, 0)),
            pl.BlockSpec((1, H), lambda i: (0, 0)),
            pl.BlockSpec((1, H), lambda i: (0, 0)),
            pl.BlockSpec((H, NT), lambda i: (0, 0)),
            pl.BlockSpec((1, NT), lambda i: (0, 0)),
        ],
        out_specs=pl.BlockSpec((S, NT), lambda i: (0, 0)),
        scratch_shapes=[pltpu.VMEM((S, D), jnp.float32)],
    )
    return pl.pallas_call(
        functools.partial(_fused_kernel, nblk=nblk),
        grid_spec=grid_spec,
        out_shape=jax.ShapeDtypeStruct((S, NT), jnp.float32),
        compiler_params=pltpu.CompilerParams(
            dimension_semantics=("arbitrary",)),
        interpret=interpret,
    )(x, bcol, blane, W_enc, b_enc.reshape(1, D), W1, b1.reshape(1, H),
      gamma1.reshape(1, H), beta1.reshape(1, H), W2, b2.reshape(1, NT))
